# baseline (speedup 1.0000x reference)
import functools

import jax
import jax.numpy as jnp
from jax import lax
from jax.experimental import pallas as pl
from jax.experimental.pallas import tpu as pltpu
from jax.experimental.pallas import tpu_sc as plsc

NC = 2
NS = 16
NW = NC * NS
L = 16
CB = 128


def _sc_mesh():
    return plsc.VectorSubcoreMesh(
        core_axis_name="c", subcore_axis_name="s", num_cores=NC, num_subcores=NS
    )


_SC_PARAMS = pltpu.CompilerParams(
    needs_layout_passes=False, use_tc_tiling_on_sc=False
)


def _deg_kernel_body(npad, ept_pad, dst_hbm, out_hbm, dst_v, acc_v):
    c = lax.axis_index("c")
    s = lax.axis_index("s")
    wid = c * NS + s
    pltpu.sync_copy(dst_hbm.at[wid], dst_v)

    zeros = jnp.zeros((L,), jnp.float32)

    def zbody(i, _):
        acc_v[pl.ds(i * L, L)] = zeros
        return 0

    lax.fori_loop(0, npad // L, zbody, 0)

    ones = jnp.ones((L,), jnp.float32)

    def ebody(i, _):
        idx = dst_v[pl.ds(i * L, L)]
        plsc.addupdate_scatter(acc_v, [idx], ones)
        return 0

    lax.fori_loop(0, ept_pad // L, ebody, 0)
    pltpu.sync_copy(acc_v, out_hbm.at[wid])


def _mp_kernel_body(npad, dpad, n_chunks, y_hbm, srci_hbm, dsti_hbm,
                    zeros_hbm, out_hbm, srcv, dstv, buf0, buf1, acc,
                    gsem0, gsem1, ssem0, ssem1):
    c = lax.axis_index("c")
    s = lax.axis_index("s")
    wid = c * NS + s
    pltpu.sync_copy(srci_hbm.at[wid], srcv)
    pltpu.sync_copy(dsti_hbm.at[wid], dstv)

    @pl.when(s == 0)
    def _():
        pltpu.sync_copy(zeros_hbm, acc)

    plsc.subcore_barrier()

    bufs = (buf0, buf1)
    gsems = (gsem0, gsem1)
    ssems = (ssem0, ssem1)

    def body(j, _):
        g = pltpu.async_copy(y_hbm.at[srcv.at[j]], buf0, gsem0)
        g.wait()
        sc = pltpu.async_copy(buf0, acc.at[dstv.at[j]], ssem0, add=True)
        sc.wait()
        return 0

    lax.fori_loop(0, n_chunks, body, 0)
    plsc.subcore_barrier()
    rpt = npad // NS
    pltpu.sync_copy(acc.at[pl.ds(s * rpt, rpt)],
                    out_hbm.at[c, pl.ds(s * rpt, rpt)])


def _s2_kernel_body(npad, ept_pad, y2_hbm, src_hbm, dst_hbm, out_hbm,
                    src_v, dst_v, y2_v, acc_v):
    c = lax.axis_index("c")
    s = lax.axis_index("s")
    wid = c * NS + s
    pltpu.sync_copy(y2_hbm, y2_v)
    pltpu.sync_copy(src_hbm.at[wid], src_v)
    pltpu.sync_copy(dst_hbm.at[wid], dst_v)

    zeros = jnp.zeros((L,), jnp.float32)

    def zbody(i, _):
        acc_v[pl.ds(i * L, L)] = zeros
        return 0

    lax.fori_loop(0, npad // L, zbody, 0)

    def ebody(i, _):
        si = src_v[pl.ds(i * L, L)]
        di = dst_v[pl.ds(i * L, L)]
        vals = plsc.load_gather(y2_v, [si])
        plsc.addupdate_scatter(acc_v, [di], vals)
        return 0

    lax.fori_loop(0, ept_pad // L, ebody, 0)
    pltpu.sync_copy(acc_v, out_hbm.at[wid])


def _k2_body(x_ref, w1_ref, degt_ref, y_ref, dinv_ref):
    deg = jnp.sum(degt_ref[...], axis=1, keepdims=True) + 1.0
    dinv = lax.rsqrt(deg)
    xw = jnp.dot(x_ref[...], w1_ref[...], preferred_element_type=jnp.float32)
    y_ref[...] = xw * dinv
    dinv_ref[...] = dinv


def _k4_body(s1_ref, y1_ref, dinv_ref, b1_ref, w2_ref, y2_ref):
    sy = s1_ref[0] + s1_ref[1] + y1_ref[...]
    h = jnp.maximum(sy * dinv_ref[...] + b1_ref[...], 0.0)
    z = jnp.dot(h, w2_ref[...], preferred_element_type=jnp.float32)
    y2_ref[...] = z * dinv_ref[...]


def _k6_body(s2t_ref, y2_ref, dinv_ref, b2_ref, out_ref):
    s2 = jnp.sum(s2t_ref[...], axis=1, keepdims=True)
    out_ref[...] = (s2 + y2_ref[...]) * dinv_ref[...] + b2_ref[...]


def kernel(x, edge_index, W1, b1, W2, b2):
    n, d_in = x.shape
    e = edge_index.shape[1]
    d_hid = W1.shape[1]
    d_out = W2.shape[1]
    f32 = jnp.float32

    npad = ((n + 1 + 127) // 128) * 128
    dpad = ((d_hid + 15) // 16) * 16
    ept = (e + NW - 1) // NW
    ept_pad = ((ept + 2 * CB - 1) // (2 * CB)) * (2 * CB)
    n_chunks = ept_pad // CB

    src = edge_index[0]
    dst = edge_index[1]
    pad_e = NW * ept - e
    if pad_e:
        src = jnp.concatenate([src, jnp.zeros((pad_e,), src.dtype)])
        dst = jnp.concatenate([dst, jnp.full((pad_e,), n, dst.dtype)])
    src2 = jnp.pad(src.reshape(NW, ept), ((0, 0), (0, ept_pad - ept)))
    dst2 = jnp.pad(dst.reshape(NW, ept), ((0, 0), (0, ept_pad - ept)),
                   constant_values=n)
    src3 = src2.reshape(NW, n_chunks, CB)
    dst3 = dst2.reshape(NW, n_chunks, CB)

    xp = jnp.pad(x, ((0, npad - n), (0, 0)))
    w1p = jnp.pad(W1, ((0, 0), (0, dpad - d_hid)))
    b1p = jnp.pad(b1, (0, dpad - d_hid)).reshape(1, dpad)
    w2p = jnp.pad(W2, ((0, dpad - d_hid), (0, 0)))
    b2p = b2.reshape(1, d_out)
    zeros_acc = jnp.zeros((npad, dpad), f32)

    mesh = _sc_mesh()

    deg_parts = pl.kernel(
        functools.partial(_deg_kernel_body, npad, ept_pad),
        out_type=jax.ShapeDtypeStruct((NW, npad), f32),
        mesh=mesh,
        compiler_params=_SC_PARAMS,
        scratch_types=[
            pltpu.VMEM((ept_pad,), jnp.int32),
            pltpu.VMEM((npad,), f32),
        ],
    )(dst2)

    y1p, dinv = pl.pallas_call(
        _k2_body,
        out_shape=[
            jax.ShapeDtypeStruct((npad, dpad), f32),
            jax.ShapeDtypeStruct((npad, 1), f32),
        ],
    )(xp, w1p, deg_parts.T)

    s1_parts = pl.kernel(
        functools.partial(_mp_kernel_body, npad, dpad, n_chunks),
        out_type=jax.ShapeDtypeStruct((NC, npad, dpad), f32),
        mesh=mesh,
        compiler_params=_SC_PARAMS,
        scratch_types=[
            pltpu.VMEM((n_chunks, CB), jnp.int32),
            pltpu.VMEM((n_chunks, CB), jnp.int32),
            pltpu.VMEM((CB, dpad), f32),
            pltpu.VMEM((CB, dpad), f32),
            pltpu.VMEM_SHARED((npad, dpad), f32),
            pltpu.SemaphoreType.DMA,
            pltpu.SemaphoreType.DMA,
            pltpu.SemaphoreType.DMA,
            pltpu.SemaphoreType.DMA,
        ],
    )(y1p, src3, dst3, zeros_acc)

    y2 = pl.pallas_call(
        _k4_body,
        out_shape=jax.ShapeDtypeStruct((npad, 1), f32),
    )(s1_parts, y1p, dinv, b1p, w2p)

    s2_parts = pl.kernel(
        functools.partial(_s2_kernel_body, npad, ept_pad),
        out_type=jax.ShapeDtypeStruct((NW, npad), f32),
        mesh=mesh,
        compiler_params=_SC_PARAMS,
        scratch_types=[
            pltpu.VMEM((ept_pad,), jnp.int32),
            pltpu.VMEM((ept_pad,), jnp.int32),
            pltpu.VMEM((npad,), f32),
            pltpu.VMEM((npad,), f32),
        ],
    )(y2.reshape(npad), src2, dst2)

    out = pl.pallas_call(
        _k6_body,
        out_shape=jax.ShapeDtypeStruct((npad, d_out), f32),
    )(s2_parts.T, y2, dinv, b2p)

    return out[:n]


# speedup vs baseline: 1.0508x; 1.0508x over previous
import functools

import jax
import jax.numpy as jnp
from jax import lax
from jax.experimental import pallas as pl
from jax.experimental.pallas import tpu as pltpu
from jax.experimental.pallas import tpu_sc as plsc

NC = 2
NS = 16
NW = NC * NS
L = 16
CB = 128


def _sc_mesh():
    return plsc.VectorSubcoreMesh(
        core_axis_name="c", subcore_axis_name="s", num_cores=NC, num_subcores=NS
    )


_SC_PARAMS = pltpu.CompilerParams(needs_layout_passes=False)

G = 8


def _deg_kernel_body(npad, ept_pad, dst_hbm, out_hbm, dst_v, acc_v):
    c = lax.axis_index("c")
    s = lax.axis_index("s")
    wid = c * NS + s
    pltpu.sync_copy(dst_hbm.at[wid], dst_v)

    zeros = jnp.zeros((L,), jnp.float32)

    def zbody(i, _):
        acc_v[pl.ds(i * L, L)] = zeros
        return 0

    lax.fori_loop(0, npad // L, zbody, 0)

    ones = jnp.ones((L,), jnp.float32)

    def ebody(i, _):
        idx = dst_v[pl.ds(i * L, L)]
        plsc.addupdate_scatter(acc_v, [idx], ones)
        return 0

    lax.fori_loop(0, ept_pad // L, ebody, 0)
    pltpu.sync_copy(acc_v, out_hbm.at[wid])


def _mp_kernel_body(npad, dpad, n_groups, y_hbm, srci_hbm, dsti_hbm,
                    zeros_hbm, out_hbm, si0, si1, di0, di1, buf0, buf1, acc,
                    isem0, isem1, gsem0, gsem1, ssem0, ssem1):
    c = lax.axis_index("c")
    s = lax.axis_index("s")
    wid = c * NS + s

    idx_slots = ((si0, di0, isem0), (si1, di1, isem1))
    bufs = (buf0, buf1)
    gsems = (gsem0, gsem1)
    ssems = (ssem0, ssem1)

    pltpu.async_copy(srci_hbm.at[wid, 0], si0, isem0)
    pltpu.async_copy(dsti_hbm.at[wid, 0], di0, isem0)

    @pl.when(s == 0)
    def _():
        pltpu.sync_copy(zeros_hbm, acc)

    plsc.subcore_barrier()
    pltpu.make_async_copy(srci_hbm.at[wid, 0], si0, isem0).wait()
    pltpu.make_async_copy(dsti_hbm.at[wid, 0], di0, isem0).wait()

    def gbody(g2, _):
        for gs in range(2):
            g = g2 * 2 + gs
            sis, dis, isem = idx_slots[gs]
            osis, odis, oisem = idx_slots[1 - gs]

            @pl.when(g > 0)
            def _():
                pltpu.make_async_copy(srci_hbm.at[wid, g], sis, isem).wait()
                pltpu.make_async_copy(dsti_hbm.at[wid, g], dis, isem).wait()

            @pl.when(g + 1 < n_groups)
            def _():
                pltpu.async_copy(srci_hbm.at[wid, g + 1], osis, oisem)
                pltpu.async_copy(dsti_hbm.at[wid, g + 1], odis, oisem)

            gdesc = [None] * G
            gdesc[0] = pltpu.async_copy(y_hbm.at[sis.at[0]], buf0, gsem0)
            gdesc[1] = pltpu.async_copy(y_hbm.at[sis.at[1]], buf1, gsem1)
            for k in range(G):
                b = k % 2
                gdesc[k].wait()
                sc = pltpu.async_copy(bufs[b], acc.at[dis.at[k]], ssems[b],
                                      add=True)
                sc.wait()
                if k + 2 < G:
                    gdesc[k + 2] = pltpu.async_copy(
                        y_hbm.at[sis.at[k + 2]], bufs[b], gsems[b])
        return 0

    lax.fori_loop(0, n_groups // 2, gbody, 0)
    plsc.subcore_barrier()
    rpt = npad // NS
    pltpu.sync_copy(acc.at[pl.ds(s * rpt, rpt)],
                    out_hbm.at[c, pl.ds(s * rpt, rpt)])


def _s2_kernel_body(npad, ept_pad, y2_hbm, src_hbm, dst_hbm, out_hbm,
                    src_v, dst_v, y2_v, acc_v):
    c = lax.axis_index("c")
    s = lax.axis_index("s")
    wid = c * NS + s
    pltpu.sync_copy(y2_hbm, y2_v)
    pltpu.sync_copy(src_hbm.at[wid], src_v)
    pltpu.sync_copy(dst_hbm.at[wid], dst_v)

    zeros = jnp.zeros((L,), jnp.float32)

    def zbody(i, _):
        acc_v[pl.ds(i * L, L)] = zeros
        return 0

    lax.fori_loop(0, npad // L, zbody, 0)

    def ebody(i, _):
        si = src_v[pl.ds(i * L, L)]
        di = dst_v[pl.ds(i * L, L)]
        vals = plsc.load_gather(y2_v, [si])
        plsc.addupdate_scatter(acc_v, [di], vals)
        return 0

    lax.fori_loop(0, ept_pad // L, ebody, 0)
    pltpu.sync_copy(acc_v, out_hbm.at[wid])


def _k2_body(x_ref, w1_ref, degt_ref, y_ref, dinv_ref):
    deg = jnp.sum(degt_ref[...], axis=1, keepdims=True) + 1.0
    dinv = lax.rsqrt(deg)
    xw = jnp.dot(x_ref[...], w1_ref[...], preferred_element_type=jnp.float32)
    y_ref[...] = xw * dinv
    dinv_ref[...] = dinv


def _k4_body(s1_ref, y1_ref, dinv_ref, b1_ref, w2_ref, y2_ref):
    sy = s1_ref[0] + s1_ref[1] + y1_ref[...]
    h = jnp.maximum(sy * dinv_ref[...] + b1_ref[...], 0.0)
    z = jnp.dot(h, w2_ref[...], preferred_element_type=jnp.float32)
    y2_ref[...] = z * dinv_ref[...]


def _k6_body(s2t_ref, y2_ref, dinv_ref, b2_ref, out_ref):
    s2 = jnp.sum(s2t_ref[...], axis=1, keepdims=True)
    out_ref[...] = (s2 + y2_ref[...]) * dinv_ref[...] + b2_ref[...]


def kernel(x, edge_index, W1, b1, W2, b2):
    n, d_in = x.shape
    e = edge_index.shape[1]
    d_hid = W1.shape[1]
    d_out = W2.shape[1]
    f32 = jnp.float32

    npad = ((n + 1 + 127) // 128) * 128
    dpad = ((d_hid + 127) // 128) * 128
    ept = (e + NW - 1) // NW
    gsz = G * CB
    n_groups = (ept + gsz - 1) // gsz
    n_groups += n_groups % 2
    ept_pad = n_groups * gsz

    src = edge_index[0]
    dst = edge_index[1]
    pad_e = NW * ept - e
    if pad_e:
        src = jnp.concatenate([src, jnp.zeros((pad_e,), src.dtype)])
        dst = jnp.concatenate([dst, jnp.full((pad_e,), n, dst.dtype)])
    src2 = jnp.pad(src.reshape(NW, ept), ((0, 0), (0, ept_pad - ept)))
    dst2 = jnp.pad(dst.reshape(NW, ept), ((0, 0), (0, ept_pad - ept)),
                   constant_values=n)
    src3 = src2.reshape(NW, n_groups, G, CB)
    dst3 = dst2.reshape(NW, n_groups, G, CB)

    xp = jnp.pad(x, ((0, npad - n), (0, 0)))
    w1p = jnp.pad(W1, ((0, 0), (0, dpad - d_hid)))
    b1p = jnp.pad(b1, (0, dpad - d_hid)).reshape(1, dpad)
    w2p = jnp.pad(W2, ((0, dpad - d_hid), (0, 0)))
    b2p = b2.reshape(1, d_out)
    zeros_acc = jnp.zeros((npad, dpad), f32)

    mesh = _sc_mesh()

    deg_parts = pl.kernel(
        functools.partial(_deg_kernel_body, npad, ept_pad),
        out_type=jax.ShapeDtypeStruct((NW, npad), f32),
        mesh=mesh,
        compiler_params=_SC_PARAMS,
        scratch_types=[
            pltpu.VMEM((ept_pad,), jnp.int32),
            pltpu.VMEM((npad,), f32),
        ],
    )(dst2)

    y1p, dinv = pl.pallas_call(
        _k2_body,
        out_shape=[
            jax.ShapeDtypeStruct((npad, dpad), f32),
            jax.ShapeDtypeStruct((npad, 1), f32),
        ],
    )(xp, w1p, deg_parts.T)

    s1_parts = pl.kernel(
        functools.partial(_mp_kernel_body, npad, dpad, n_groups),
        out_type=jax.ShapeDtypeStruct((NC, npad, dpad), f32),
        mesh=mesh,
        compiler_params=_SC_PARAMS,
        scratch_types=[
            pltpu.VMEM((G, CB), jnp.int32),
            pltpu.VMEM((G, CB), jnp.int32),
            pltpu.VMEM((G, CB), jnp.int32),
            pltpu.VMEM((G, CB), jnp.int32),
            pltpu.VMEM((CB, dpad), f32),
            pltpu.VMEM((CB, dpad), f32),
            pltpu.VMEM_SHARED((npad, dpad), f32),
            pltpu.SemaphoreType.DMA,
            pltpu.SemaphoreType.DMA,
            pltpu.SemaphoreType.DMA,
            pltpu.SemaphoreType.DMA,
            pltpu.SemaphoreType.DMA,
            pltpu.SemaphoreType.DMA,
        ],
    )(y1p, src3, dst3, zeros_acc)

    y2 = pl.pallas_call(
        _k4_body,
        out_shape=jax.ShapeDtypeStruct((npad, 1), f32),
    )(s1_parts, y1p, dinv, b1p, w2p)

    s2_parts = pl.kernel(
        functools.partial(_s2_kernel_body, npad, ept_pad),
        out_type=jax.ShapeDtypeStruct((NW, npad), f32),
        mesh=mesh,
        compiler_params=_SC_PARAMS,
        scratch_types=[
            pltpu.VMEM((ept_pad,), jnp.int32),
            pltpu.VMEM((ept_pad,), jnp.int32),
            pltpu.VMEM((npad,), f32),
            pltpu.VMEM((npad,), f32),
        ],
    )(y2.reshape(npad), src2, dst2)

    out = pl.pallas_call(
        _k6_body,
        out_shape=jax.ShapeDtypeStruct((npad, d_out), f32),
    )(s2_parts.T, y2, dinv, b2p)

    return out[:n]


# speedup vs baseline: 1.3164x; 1.2527x over previous
import functools

import jax
import jax.numpy as jnp
from jax import lax
from jax.experimental import pallas as pl
from jax.experimental.pallas import tpu as pltpu
from jax.experimental.pallas import tpu_sc as plsc

NC = 2
NS = 16
NW = NC * NS
L = 16
CB = 128


def _sc_mesh():
    return plsc.VectorSubcoreMesh(
        core_axis_name="c", subcore_axis_name="s", num_cores=NC, num_subcores=NS
    )


_SC_PARAMS = pltpu.CompilerParams(needs_layout_passes=False)


def _deg_kernel_body(npad, ept_pad, dst_hbm, out_hbm, dst_v, acc_v):
    c = lax.axis_index("c")
    s = lax.axis_index("s")
    wid = c * NS + s
    pltpu.sync_copy(dst_hbm.at[wid], dst_v)

    zeros = jnp.zeros((L,), jnp.float32)

    def zbody(i, _):
        acc_v[pl.ds(i * L, L)] = zeros
        return 0

    lax.fori_loop(0, npad // L, zbody, 0)

    ones = jnp.ones((L,), jnp.float32)

    def ebody(i, _):
        idx = dst_v[pl.ds(i * L, L)]
        plsc.addupdate_scatter(acc_v, [idx], ones)
        return 0

    lax.fori_loop(0, ept_pad // L, ebody, 0)
    pltpu.sync_copy(acc_v, out_hbm.at[wid])


def _mp_kernel_body(npad, dpad, n_chunks, y_hbm, srci_hbm, dsti_hbm,
                    zeros_hbm, out_hbm, srcv, dstv, buf, acc, gsem, ssem):
    c = lax.axis_index("c")
    s = lax.axis_index("s")
    wid = c * NS + s
    pltpu.sync_copy(srci_hbm.at[wid], srcv)
    pltpu.sync_copy(dsti_hbm.at[wid], dstv)

    @pl.when(s == 0)
    def _():
        pltpu.sync_copy(zeros_hbm, acc)

    plsc.subcore_barrier()

    def body(j, _):
        g = pltpu.async_copy(y_hbm.at[srcv.at[j]], buf, gsem)
        g.wait()
        sc = pltpu.async_copy(buf, acc.at[dstv.at[j]], ssem, add=True)
        sc.wait()
        return 0

    lax.fori_loop(0, n_chunks, body, 0)
    plsc.subcore_barrier()
    rpt = npad // NS
    pltpu.sync_copy(acc.at[pl.ds(s * rpt, rpt)],
                    out_hbm.at[c, pl.ds(s * rpt, rpt)])


def _s2_kernel_body(npad, ept_pad, y2_hbm, src_hbm, dst_hbm, out_hbm,
                    src_v, dst_v, y2_v, acc_v):
    c = lax.axis_index("c")
    s = lax.axis_index("s")
    wid = c * NS + s
    pltpu.sync_copy(y2_hbm, y2_v)
    pltpu.sync_copy(src_hbm.at[wid], src_v)
    pltpu.sync_copy(dst_hbm.at[wid], dst_v)

    zeros = jnp.zeros((L,), jnp.float32)

    def zbody(i, _):
        acc_v[pl.ds(i * L, L)] = zeros
        return 0

    lax.fori_loop(0, npad // L, zbody, 0)

    def ebody(i, _):
        si = src_v[pl.ds(i * L, L)]
        di = dst_v[pl.ds(i * L, L)]
        vals = plsc.load_gather(y2_v, [si])
        plsc.addupdate_scatter(acc_v, [di], vals)
        return 0

    lax.fori_loop(0, ept_pad // L, ebody, 0)
    pltpu.sync_copy(acc_v, out_hbm.at[wid])


def _k2_body(x_ref, w1_ref, degt_ref, y_ref, dinv_ref):
    deg = jnp.sum(degt_ref[...], axis=1, keepdims=True) + 1.0
    dinv = lax.rsqrt(deg)
    xw = jnp.dot(x_ref[...], w1_ref[...], preferred_element_type=jnp.float32)
    y_ref[...] = xw * dinv
    dinv_ref[...] = dinv


def _k4_body(s1_ref, y1_ref, dinv_ref, b1_ref, w2_ref, y2_ref):
    sy = s1_ref[0] + s1_ref[1] + y1_ref[...]
    h = jnp.maximum(sy * dinv_ref[...] + b1_ref[...], 0.0)
    z = jnp.dot(h, w2_ref[...], preferred_element_type=jnp.float32)
    y2_ref[...] = z * dinv_ref[...]


def _k6_body(s2t_ref, y2_ref, dinv_ref, b2_ref, out_ref):
    s2 = jnp.sum(s2t_ref[...], axis=1, keepdims=True)
    out_ref[...] = (s2 + y2_ref[...]) * dinv_ref[...] + b2_ref[...]


def kernel(x, edge_index, W1, b1, W2, b2):
    n, d_in = x.shape
    e = edge_index.shape[1]
    d_hid = W1.shape[1]
    d_out = W2.shape[1]
    f32 = jnp.float32

    npad = ((n + 1 + 127) // 128) * 128
    dpad = ((d_hid + 127) // 128) * 128
    ept = (e + NW - 1) // NW
    ept_pad = ((ept + CB - 1) // CB) * CB
    n_chunks = ept_pad // CB

    src = edge_index[0]
    dst = edge_index[1]
    pad_e = NW * ept - e
    if pad_e:
        src = jnp.concatenate([src, jnp.zeros((pad_e,), src.dtype)])
        dst = jnp.concatenate([dst, jnp.full((pad_e,), n, dst.dtype)])
    src2 = jnp.pad(src.reshape(NW, ept), ((0, 0), (0, ept_pad - ept)))
    dst2 = jnp.pad(dst.reshape(NW, ept), ((0, 0), (0, ept_pad - ept)),
                   constant_values=n)
    src3 = src2.reshape(NW, n_chunks, CB)
    dst3 = dst2.reshape(NW, n_chunks, CB)

    xp = jnp.pad(x, ((0, npad - n), (0, 0)))
    w1p = jnp.pad(W1, ((0, 0), (0, dpad - d_hid)))
    b1p = jnp.pad(b1, (0, dpad - d_hid)).reshape(1, dpad)
    w2p = jnp.pad(W2, ((0, dpad - d_hid), (0, 0)))
    b2p = b2.reshape(1, d_out)
    zeros_acc = jnp.zeros((npad, dpad), f32)

    mesh = _sc_mesh()

    deg_parts = pl.kernel(
        functools.partial(_deg_kernel_body, npad, ept_pad),
        out_type=jax.ShapeDtypeStruct((NW, npad), f32),
        mesh=mesh,
        compiler_params=_SC_PARAMS,
        scratch_types=[
            pltpu.VMEM((ept_pad,), jnp.int32),
            pltpu.VMEM((npad,), f32),
        ],
    )(dst2)

    y1p, dinv = pl.pallas_call(
        _k2_body,
        out_shape=[
            jax.ShapeDtypeStruct((npad, dpad), f32),
            jax.ShapeDtypeStruct((npad, 1), f32),
        ],
    )(xp, w1p, deg_parts.T)

    s1_parts = pl.kernel(
        functools.partial(_mp_kernel_body, npad, dpad, n_chunks),
        out_type=jax.ShapeDtypeStruct((NC, npad, dpad), f32),
        mesh=mesh,
        compiler_params=_SC_PARAMS,
        scratch_types=[
            pltpu.VMEM((n_chunks, CB), jnp.int32),
            pltpu.VMEM((n_chunks, CB), jnp.int32),
            pltpu.VMEM((CB, dpad), f32),
            pltpu.VMEM_SHARED((npad, dpad), f32),
            pltpu.SemaphoreType.DMA,
            pltpu.SemaphoreType.DMA,
        ],
    )(y1p, src3, dst3, zeros_acc)

    y2 = pl.pallas_call(
        _k4_body,
        out_shape=jax.ShapeDtypeStruct((npad, 1), f32),
    )(s1_parts, y1p, dinv, b1p, w2p)

    s2_parts = pl.kernel(
        functools.partial(_s2_kernel_body, npad, ept_pad),
        out_type=jax.ShapeDtypeStruct((NW, npad), f32),
        mesh=mesh,
        compiler_params=_SC_PARAMS,
        scratch_types=[
            pltpu.VMEM((ept_pad,), jnp.int32),
            pltpu.VMEM((ept_pad,), jnp.int32),
            pltpu.VMEM((npad,), f32),
            pltpu.VMEM((npad,), f32),
        ],
    )(y2.reshape(npad), src2, dst2)

    out = pl.pallas_call(
        _k6_body,
        out_shape=jax.ShapeDtypeStruct((npad, d_out), f32),
    )(s2_parts.T, y2, dinv, b2p)

    return out[:n]
